# baseline (speedup 1.0000x reference)
import functools

import jax
import jax.numpy as jnp
from jax import lax
from jax.experimental import pallas as pl
from jax.experimental.pallas import tpu as pltpu
from jax.experimental.pallas import tpu_sc as plsc

TOP_K = 2
N_EXPERTS = 8
NEG = -1e30
LANES = 16


def _logits_body(x_ref, w_ref, lt_ref):
    x = x_ref[...]
    w = w_ref[...]
    lt_ref[...] = jnp.dot(x, w, preferred_element_type=jnp.float32).T


@functools.partial(jax.jit, static_argnames=("block_m",))
def _logits_t(x, wp, block_m):
    tokens, h = x.shape
    grid = tokens // block_m
    return pl.pallas_call(
        _logits_body,
        grid=(grid,),
        in_specs=[
            pl.BlockSpec((block_m, h), lambda i: (i, 0)),
            pl.BlockSpec((h, N_EXPERTS), lambda i: (0, 0)),
        ],
        out_specs=pl.BlockSpec((N_EXPERTS, block_m), lambda i: (0, i)),
        out_shape=jax.ShapeDtypeStruct((N_EXPERTS, tokens), jnp.float32),
        compiler_params=pltpu.CompilerParams(
            dimension_semantics=("arbitrary",),
        ),
    )(x, wp)


@functools.cache
def _make_route(tokens):
    info = plsc.get_sparse_core_info()
    nw = info.num_cores * info.num_subcores
    chunk = tokens // nw
    mesh = plsc.VectorSubcoreMesh(core_axis_name="c", subcore_axis_name="s")

    @functools.partial(
        pl.kernel,
        out_type=[
            jax.ShapeDtypeStruct((TOP_K, tokens), jnp.int32),
            jax.ShapeDtypeStruct((TOP_K, tokens), jnp.float32),
        ],
        mesh=mesh,
        scratch_types=[
            pltpu.VMEM((N_EXPERTS, chunk), jnp.float32),
            pltpu.VMEM((TOP_K, chunk), jnp.int32),
            pltpu.VMEM((TOP_K, chunk), jnp.float32),
        ],
    )
    def route(lt_hbm, idx_hbm, wgt_hbm, lt_v, idx_v, wgt_v):
        wid = lax.axis_index("s") * info.num_cores + lax.axis_index("c")
        base = wid * chunk
        pltpu.sync_copy(lt_hbm.at[:, pl.ds(base, chunk)], lt_v)

        def body(j, carry):
            sl = pl.ds(j * LANES, LANES)
            rows = [lt_v[e, sl] for e in range(N_EXPERTS)]
            m1 = rows[0]
            for e in range(1, N_EXPERTS):
                m1 = jnp.maximum(m1, rows[e])
            i1 = jnp.full((LANES,), N_EXPERTS - 1, jnp.int32)
            for e in range(N_EXPERTS - 2, -1, -1):
                i1 = jnp.where(rows[e] == m1, e, i1)
            l2 = [jnp.where(i1 == e, NEG, rows[e]) for e in range(N_EXPERTS)]
            m2 = l2[0]
            for e in range(1, N_EXPERTS):
                m2 = jnp.maximum(m2, l2[e])
            i2 = jnp.full((LANES,), N_EXPERTS - 1, jnp.int32)
            for e in range(N_EXPERTS - 2, -1, -1):
                i2 = jnp.where(l2[e] == m2, e, i2)
            ex = jnp.exp(m2 - m1)
            w1 = 1.0 / (1.0 + ex)
            w2 = ex * w1
            idx_v[0, sl] = i1
            idx_v[1, sl] = i2
            wgt_v[0, sl] = w1
            wgt_v[1, sl] = w2
            return carry

        lax.fori_loop(0, chunk // LANES, body, 0)
        pltpu.sync_copy(idx_v, idx_hbm.at[:, pl.ds(base, chunk)])
        pltpu.sync_copy(wgt_v, wgt_hbm.at[:, pl.ds(base, chunk)])

    return route


def kernel(hidden_states, weight):
    bsz, seq_len, h = hidden_states.shape
    x = hidden_states.reshape(bsz * seq_len, h)
    lt = _logits_t(x, weight.T, 4096)
    idx_t, wgt_t = _make_route(bsz * seq_len)(lt)
    topk_idx = idx_t.T
    topk_weight = wgt_t.T
    aux_loss = jnp.zeros((), jnp.float32)
    return topk_idx, topk_weight, aux_loss


# speedup vs baseline: 1.3875x; 1.3875x over previous
import functools

import jax
import jax.numpy as jnp
from jax.experimental import pallas as pl
from jax.experimental.pallas import tpu as pltpu

TOP_K = 2
N_EXPERTS = 8
NEG = -1e30


def _gate_body(x_ref, w_ref, idx_ref, wgt_ref, acc_ref):
    k = pl.program_id(1)
    part = jnp.dot(x_ref[...], w_ref[...], preferred_element_type=jnp.float32).T

    @pl.when(k == 0)
    def _():
        acc_ref[...] = part

    @pl.when(k == 1)
    def _():
        _finish(acc_ref[...] + part, idx_ref, wgt_ref)


def _finish(lt, idx_ref, wgt_ref):
    row = jax.lax.broadcasted_iota(jnp.int32, lt.shape, 0)
    m1 = jnp.max(lt, axis=0, keepdims=True)
    i1 = jnp.min(jnp.where(lt == m1, row, N_EXPERTS), axis=0, keepdims=True)
    l2 = jnp.where(row == i1, NEG, lt)
    m2 = jnp.max(l2, axis=0, keepdims=True)
    i2 = jnp.min(jnp.where(l2 == m2, row, N_EXPERTS), axis=0, keepdims=True)
    e = jnp.exp(m2 - m1)
    w1 = 1.0 / (1.0 + e)
    w2 = e * w1
    idx_ref[...] = jnp.concatenate([i1, i2], axis=0)
    wgt_ref[...] = jnp.concatenate([w1, w2], axis=0)


@functools.partial(jax.jit, static_argnames=("block_m",))
def _gate(x, wp, block_m):
    tokens, h = x.shape
    grid = tokens // block_m
    return pl.pallas_call(
        _gate_body,
        grid=(grid, 2),
        in_specs=[
            pl.BlockSpec((block_m, h // 2), lambda i, k: (i, k)),
            pl.BlockSpec((h // 2, N_EXPERTS), lambda i, k: (k, 0)),
        ],
        out_specs=[
            pl.BlockSpec((TOP_K, block_m), lambda i, k: (0, i)),
            pl.BlockSpec((TOP_K, block_m), lambda i, k: (0, i)),
        ],
        out_shape=[
            jax.ShapeDtypeStruct((TOP_K, tokens), jnp.int32),
            jax.ShapeDtypeStruct((TOP_K, tokens), jnp.float32),
        ],
        scratch_shapes=[pltpu.VMEM((N_EXPERTS, block_m), jnp.float32)],
        compiler_params=pltpu.CompilerParams(
            dimension_semantics=("arbitrary", "arbitrary"),
        ),
    )(x, wp)


def kernel(hidden_states, weight):
    bsz, seq_len, h = hidden_states.shape
    x = hidden_states.reshape(bsz * seq_len, h)
    idx_t, wgt_t = _gate(x, weight.T, 4096)
    topk_idx = idx_t.T
    topk_weight = wgt_t.T
    aux_loss = jnp.zeros((), jnp.float32)
    return topk_idx, topk_weight, aux_loss


# speedup vs baseline: 1.5352x; 1.1065x over previous
import functools

import jax
import jax.numpy as jnp
from jax.experimental import pallas as pl
from jax.experimental.pallas import tpu as pltpu

TOP_K = 2
N_EXPERTS = 8
NEG = -1e30


def _gate_body(x_ref, w_ref, idx_ref, wgt_ref):
    x = x_ref[...]
    w = w_ref[...]
    logits = jnp.dot(x, w, preferred_element_type=jnp.float32)
    lt = logits.T
    row = jax.lax.broadcasted_iota(jnp.int32, lt.shape, 0)
    m1 = jnp.max(lt, axis=0, keepdims=True)
    i1 = jnp.min(jnp.where(lt == m1, row, N_EXPERTS), axis=0, keepdims=True)
    l2 = jnp.where(row == i1, NEG, lt)
    m2 = jnp.max(l2, axis=0, keepdims=True)
    i2 = jnp.min(jnp.where(l2 == m2, row, N_EXPERTS), axis=0, keepdims=True)
    e = jnp.exp(m2 - m1)
    w1 = 1.0 / (1.0 + e)
    w2 = e * w1
    idx_ref[...] = jnp.concatenate([i1, i2], axis=0)
    wgt_ref[...] = jnp.concatenate([w1, w2], axis=0)


@functools.partial(jax.jit, static_argnames=("block_m",))
def _gate(x, wp, block_m):
    tokens, h = x.shape
    grid = tokens // block_m
    return pl.pallas_call(
        _gate_body,
        grid=(grid,),
        in_specs=[
            pl.BlockSpec((block_m, h), lambda i: (i, 0)),
            pl.BlockSpec((h, N_EXPERTS), lambda i: (0, 0)),
        ],
        out_specs=[
            pl.BlockSpec((TOP_K, block_m), lambda i: (0, i)),
            pl.BlockSpec((TOP_K, block_m), lambda i: (0, i)),
        ],
        out_shape=[
            jax.ShapeDtypeStruct((TOP_K, tokens), jnp.int32),
            jax.ShapeDtypeStruct((TOP_K, tokens), jnp.float32),
        ],
        compiler_params=pltpu.CompilerParams(
            dimension_semantics=("arbitrary",),
        ),
    )(x, wp)


def kernel(hidden_states, weight):
    bsz, seq_len, h = hidden_states.shape
    x = hidden_states.reshape(bsz * seq_len, h)
    idx_t, wgt_t = _gate(x, weight.T, 4096)
    topk_idx = idx_t.T
    topk_weight = wgt_t.T
    aux_loss = jnp.zeros((), jnp.float32)
    return topk_idx, topk_weight, aux_loss
